# baseline (speedup 1.0000x reference)
import jax
import jax.numpy as jnp
from jax.experimental import pallas as pl
from jax.experimental.pallas import tpu as pltpu

_N = 1048576
_ROWS = _N // 128
_NPROG = 2
_PROWS = _ROWS // _NPROG
_CHUNK = 1024
_NCHUNK = _PROWS // _CHUNK


def _tc_body(p_hbm, m_hbm, o_ref, pbuf, mbuf, psem, msem):
    base = pl.program_id(0) * _PROWS
    for c in range(_NCHUNK):
        pltpu.make_async_copy(
            p_hbm.at[pl.ds(base + c * _CHUNK, _CHUNK), :],
            pbuf.at[c], psem.at[c]
        ).start()
        pltpu.make_async_copy(
            m_hbm.at[pl.ds(base + c * _CHUNK, _CHUNK), :],
            mbuf.at[c], msem.at[c]
        ).start()

    acc = None
    for c in range(_NCHUNK):
        pltpu.make_async_copy(
            p_hbm.at[pl.ds(base + c * _CHUNK, _CHUNK), :],
            pbuf.at[c], psem.at[c]
        ).wait()
        pltpu.make_async_copy(
            m_hbm.at[pl.ds(base + c * _CHUNK, _CHUNK), :],
            mbuf.at[c], msem.at[c]
        ).wait()
        for k in range(0, _CHUNK, 64):
            m = mbuf[c, pl.ds(k, 64), :]
            p = pbuf[c, pl.ds(k, 64), :]
            pm = jnp.where(m != 0.0, p, 0.0)
            contrib = jnp.exp(-0.5 * m * m) * pm
            acc = contrib if acc is None else acc + contrib

    while acc.shape[0] > 8:
        h = acc.shape[0] // 2
        acc = acc[:h] + acc[h:]
    s = jnp.sum(acc) * (-1.0 / _N)
    o_ref[...] = jnp.full((8, 128), s, jnp.float32)


def kernel(preds, margin):
    p2 = preds.reshape(_ROWS, 128)
    m2 = margin.reshape(_ROWS, 128)
    out = pl.pallas_call(
        _tc_body,
        grid=(_NPROG,),
        in_specs=[
            pl.BlockSpec(memory_space=pl.ANY),
            pl.BlockSpec(memory_space=pl.ANY),
        ],
        out_specs=pl.BlockSpec((8, 128), lambda i: (i, 0)),
        out_shape=jax.ShapeDtypeStruct((_NPROG * 8, 128), jnp.float32),
        scratch_shapes=[
            pltpu.VMEM((_NCHUNK, _CHUNK, 128), jnp.float32),
            pltpu.VMEM((_NCHUNK, _CHUNK, 128), jnp.float32),
            pltpu.SemaphoreType.DMA((_NCHUNK,)),
            pltpu.SemaphoreType.DMA((_NCHUNK,)),
        ],
        compiler_params=pltpu.CompilerParams(
            dimension_semantics=("parallel",),
        ),
    )(p2, m2)
    return out[0, 0] + out[8, 0]


# speedup vs baseline: 1.9639x; 1.9639x over previous
import jax
import jax.numpy as jnp
from jax.experimental import pallas as pl
from jax.experimental.pallas import tpu as pltpu

_N = 1048576
_ROWS = _N // 128
_SIZES = (1024, 1024, 1024, 1024, 1024, 1024, 1024, 1024)
_STARTS = tuple(sum(_SIZES[:i]) for i in range(len(_SIZES)))
_NCHUNK = len(_SIZES)
_BUFROWS = max(_SIZES)


def _tc_body(p_hbm, m_hbm, o_ref, pbuf, mbuf, psem, msem):
    for c in range(_NCHUNK):
        pltpu.make_async_copy(
            p_hbm.at[pl.ds(_STARTS[c], _SIZES[c]), :],
            pbuf.at[c, pl.ds(0, _SIZES[c]), :], psem.at[c]
        ).start()
        pltpu.make_async_copy(
            m_hbm.at[pl.ds(_STARTS[c], _SIZES[c]), :],
            mbuf.at[c, pl.ds(0, _SIZES[c]), :], msem.at[c]
        ).start()

    acc = None
    for c in range(_NCHUNK):
        pltpu.make_async_copy(
            p_hbm.at[pl.ds(_STARTS[c], _SIZES[c]), :],
            pbuf.at[c, pl.ds(0, _SIZES[c]), :], psem.at[c]
        ).wait()
        pltpu.make_async_copy(
            m_hbm.at[pl.ds(_STARTS[c], _SIZES[c]), :],
            mbuf.at[c, pl.ds(0, _SIZES[c]), :], msem.at[c]
        ).wait()
        for k in range(0, _SIZES[c], 64):
            m = mbuf[c, pl.ds(k, 64), :]
            p = pbuf[c, pl.ds(k, 64), :]
            pm = jnp.where(m != 0.0, p, 0.0)
            contrib = jnp.exp(-0.5 * m * m) * pm
            acc = contrib if acc is None else acc + contrib

    while acc.shape[0] > 8:
        h = acc.shape[0] // 2
        acc = acc[:h] + acc[h:]
    o_ref[0, 0] = jnp.sum(acc) * (-1.0 / _N)


def kernel(preds, margin):
    p2 = preds.reshape(_ROWS, 128)
    m2 = margin.reshape(_ROWS, 128)
    out = pl.pallas_call(
        _tc_body,
        in_specs=[
            pl.BlockSpec(memory_space=pl.ANY),
            pl.BlockSpec(memory_space=pl.ANY),
        ],
        out_specs=pl.BlockSpec(memory_space=pltpu.SMEM),
        out_shape=jax.ShapeDtypeStruct((1, 1), jnp.float32),
        scratch_shapes=[
            pltpu.VMEM((_NCHUNK, _BUFROWS, 128), jnp.float32),
            pltpu.VMEM((_NCHUNK, _BUFROWS, 128), jnp.float32),
            pltpu.SemaphoreType.DMA((_NCHUNK,)),
            pltpu.SemaphoreType.DMA((_NCHUNK,)),
        ],
    )(p2, m2)
    return out[0, 0]
